# baseline (speedup 1.0000x reference)
import functools

import jax
import jax.numpy as jnp
from jax import lax
from jax.experimental import pallas as pl
from jax.experimental.pallas import tpu as pltpu
from jax.experimental.pallas import tpu_sc as plsc

L = 16
NS = 16
BIG = 1e30
BIGI = 1 << 30


def _iota():
  return lax.iota(jnp.int32, L)


def _bcast(x, dtype):
  return jnp.full((L,), x, dtype=dtype)


def _shuf(x, sh):
  perm = jnp.bitwise_xor(_iota(), sh)
  return jnp.take_along_axis(x, perm, axis=0)


def _vmin(x):
  for sh in (1, 2, 4, 8):
    x = jnp.minimum(x, _shuf(x, sh))
  return x


def _vsum(x):
  for sh in (1, 2, 4, 8):
    x = x + _shuf(x, sh)
  return x


def _sc_body(npad, kpad, ngt, gpt, ppt, scale,
             px_h, py_h, ps_h, gx4_h, gy4_h, glab_h,
             ai_h, al_h, loss_h,
             px_v, py_v, ps_v, gx4_v, gy4_v, glab_v,
             gxv, gyv, gwv, ghv, gsv,
             mdrow_v, mirow_v, smd, smi, mdg_v, mig_v,
             ai_v, al_v, loss_v):
  sid = lax.axis_index("s")
  n_chunks = npad // L
  iot = _iota()

  pltpu.sync_copy(px_h, px_v)
  pltpu.sync_copy(py_h, py_v)
  pltpu.sync_copy(ps_h, ps_v)
  pltpu.sync_copy(gx4_h, gx4_v)
  pltpu.sync_copy(gy4_h, gy4_v)
  pltpu.sync_copy(glab_h, glab_v)

  def mm_body(c, carry):
    smin, smax = carry
    s = ps_v[pl.ds(sid * ppt + c * L, L)]
    real = s > 0.0
    smin = jnp.minimum(smin, jnp.where(real, s, BIG))
    smax = jnp.maximum(smax, jnp.where(real, s, -BIG))
    return smin, smax

  smin, smax = lax.fori_loop(
      0, ppt // L, mm_body,
      (jnp.full((L,), BIG, jnp.float32), jnp.full((L,), -BIG, jnp.float32)))
  mdrow_v[...] = jnp.where(iot < 8, _vmin(smin), _vmin(jnp.negative(smax)))
  pltpu.sync_copy(mdrow_v, smd.at[sid])
  plsc.subcore_barrier()
  pltpu.sync_copy(smd, mdg_v)
  plsc.subcore_barrier()

  def amerge_body(t, acc):
    row = plsc.load_gather(mdg_v, [_bcast(t, jnp.int32), iot])
    return jnp.minimum(acc, row)

  acc = lax.fori_loop(0, NS, amerge_body, jnp.full((L,), BIG, jnp.float32))
  smin_b = _vmin(jnp.where(iot < 8, acc, BIG))
  smax_b = jnp.negative(_vmin(jnp.where(iot < 8, BIG, acc)))

  for c in range(kpad // L):
    o = c * L
    xs0 = gx4_v[pl.ds(0 * kpad + o, L)]
    xs1 = gx4_v[pl.ds(1 * kpad + o, L)]
    xs2 = gx4_v[pl.ds(2 * kpad + o, L)]
    xs3 = gx4_v[pl.ds(3 * kpad + o, L)]
    ys0 = gy4_v[pl.ds(0 * kpad + o, L)]
    ys1 = gy4_v[pl.ds(1 * kpad + o, L)]
    ys2 = gy4_v[pl.ds(2 * kpad + o, L)]
    ys3 = gy4_v[pl.ds(3 * kpad + o, L)]
    x1 = jnp.minimum(jnp.minimum(xs0, xs1), jnp.minimum(xs2, xs3))
    x2 = jnp.maximum(jnp.maximum(xs0, xs1), jnp.maximum(xs2, xs3))
    y1 = jnp.minimum(jnp.minimum(ys0, ys1), jnp.minimum(ys2, ys3))
    y2 = jnp.maximum(jnp.maximum(ys0, ys1), jnp.maximum(ys2, ys3))
    w = jnp.maximum(x2 - x1, 1e-6)
    h = jnp.maximum(y2 - y1, 1e-6)
    p = (w * h) * (1.0 / (scale * scale))
    glvl = jnp.zeros((L,), jnp.int32)
    thr = 4.0
    for _ in range(12):
      glvl = glvl + jnp.where(p >= thr, 1, 0).astype(jnp.int32)
      thr = thr * 4.0
    gs = (jnp.int32(1) << glvl).astype(jnp.float32)
    gs = jnp.clip(gs, smin_b, smax_b)
    valid = (iot + o) < ngt
    gs = jnp.where(valid, gs, -2.0)
    gxv[pl.ds(o, L)] = (x1 + x2) * 0.5
    gyv[pl.ds(o, L)] = (y1 + y2) * 0.5
    gwv[pl.ds(o, L)] = 1.0 / w
    ghv[pl.ds(o, L)] = 1.0 / h
    gsv[pl.ds(o, L)] = gs

  prm = []
  for j in range(gpt):
    gsp = _bcast(sid * gpt + j, jnp.int32)
    prm.append((plsc.load_gather(gxv, [gsp]),
                plsc.load_gather(gyv, [gsp]),
                plsc.load_gather(gwv, [gsp]),
                plsc.load_gather(ghv, [gsp]),
                plsc.load_gather(gsv, [gsp])))

  init = tuple(jnp.full((L,), BIG, jnp.float32) for _ in range(gpt)) + \
      tuple(jnp.zeros((L,), jnp.int32) for _ in range(gpt))

  @plsc.parallel_loop(0, n_chunks, 1, unroll=2, carry=init)
  def pt_loop(c, pc):
    ds = list(pc[:gpt])
    inds = list(pc[gpt:])
    base = c * L
    px = px_v[pl.ds(base, L)]
    py = py_v[pl.ds(base, L)]
    ps = ps_v[pl.ds(base, L)]
    idx = iot + _bcast(base, jnp.int32)
    for j in range(gpt):
      gx_b, gy_b, gw_b, gh_b, gs_b = prm[j]
      dx = (px - gx_b) * gw_b
      dy = (py - gy_b) * gh_b
      d2 = dx * dx + dy * dy
      upd = (d2 < ds[j]) & (ps == gs_b)
      ds[j] = jnp.where(upd, d2, ds[j])
      inds[j] = jnp.where(upd, idx, inds[j])
    return tuple(ds) + tuple(inds)

  md_vec = jnp.full((L,), BIG, jnp.float32)
  mi_vec = jnp.zeros((L,), jnp.int32)
  for j in range(gpt):
    run_d2 = pt_loop[j]
    run_idx = pt_loop[gpt + j]
    md_b = _vmin(run_d2)
    cand = jnp.where(run_d2 == md_b, run_idx, BIGI)
    mi_b = _vmin(cand)
    lane = iot == _bcast(j, jnp.int32)
    md_vec = jnp.where(lane, md_b, md_vec)
    mi_vec = jnp.where(lane, mi_b, mi_vec)

  mdrow_v[...] = md_vec
  mirow_v[...] = mi_vec
  pltpu.sync_copy(mdrow_v, smd.at[sid])
  pltpu.sync_copy(mirow_v, smi.at[sid])
  plsc.subcore_barrier()
  pltpu.sync_copy(smd, mdg_v)
  pltpu.sync_copy(smi, mig_v)

  for c in range(ppt // L):
    ai_v[pl.ds(c * L, L)] = jnp.zeros((L,), jnp.int32)
    al_v[pl.ds(c * L, L)] = jnp.zeros((L,), jnp.int32)

  base = sid * ppt
  base_b = _bcast(base, jnp.int32)
  lsum = jnp.zeros((L,), jnp.float32)
  lcnt = jnp.zeros((L,), jnp.float32)
  for r in range(gpt):
    k_vec = iot * gpt + r
    rsp = _bcast(r, jnp.int32)
    md_col = plsc.load_gather(mdg_v, [iot, rsp])
    mi_col = plsc.load_gather(mig_v, [iot, rsp])
    valid_k = md_col < (BIG * 0.5)

    def kill_body(jt, killed):
      jtsp = _bcast(jt, jnp.int32)
      k = killed
      for jj in range(gpt):
        jjsp = _bcast(jj, jnp.int32)
        md_j = plsc.load_gather(mdg_v, [jtsp, jjsp])
        mi_j = plsc.load_gather(mig_v, [jtsp, jjsp])
        g_j = jt * gpt + jj
        g_j_b = _bcast(g_j, jnp.int32)
        better = (md_j < md_col) | ((md_j == md_col) & (g_j_b < k_vec))
        k = k | ((md_j < (BIG * 0.5)) & (mi_j == mi_col) & better)
      return k

    killed = lax.fori_loop(0, NS, kill_body, jnp.zeros((L,), jnp.bool_))
    win = valid_k & jnp.logical_not(killed)

    lsum = lsum + jnp.where(win, md_col, 0.0)
    lcnt = lcnt + jnp.where(win, 1.0, 0.0)

    mine = win & (mi_col >= base_b) & (mi_col < base_b + ppt)
    loc = mi_col - base_b
    loc = jnp.where(mine, loc, 0)
    plsc.store_scatter(ai_v, [loc], k_vec + 1, mask=mine)
    labs = plsc.load_gather(glab_v, [k_vec])
    plsc.store_scatter(al_v, [loc], labs, mask=mine)

  total = _vsum(lsum)
  cnt = _vsum(lcnt)
  loss_v[...] = total / jnp.maximum(cnt, 1.0)

  pltpu.sync_copy(ai_v, ai_h.at[pl.ds(base, ppt)])
  pltpu.sync_copy(al_v, al_h.at[pl.ds(base, ppt)])

  @pl.when(sid == 0)
  def _():
    pltpu.sync_copy(loss_v, loss_h)


@jax.jit
def kernel(points, gt_obboxes, gt_labels):
  n = points.shape[0]
  k = gt_obboxes.shape[0]
  scale = 4.0

  npad = ((n + L * NS - 1) // (L * NS)) * (L * NS)
  ppt = npad // NS
  gpt = (k + NS - 1) // NS
  kpad = gpt * NS

  px = jnp.pad(points[:, 0], (0, npad - n))
  py = jnp.pad(points[:, 1], (0, npad - n))
  ps = jnp.pad(points[:, 2], (0, npad - n), constant_values=-1.0)
  gx4 = jnp.pad(gt_obboxes[:, 0::2].T, ((0, 0), (0, kpad - k)),
                constant_values=1.0).reshape(-1)
  gy4 = jnp.pad(gt_obboxes[:, 1::2].T, ((0, 0), (0, kpad - k)),
                constant_values=1.0).reshape(-1)
  glab = jnp.pad(gt_labels.astype(jnp.int32), (0, kpad - k))

  mesh = plsc.VectorSubcoreMesh(
      core_axis_name="c", subcore_axis_name="s", num_cores=1, num_subcores=NS)

  body = functools.partial(_sc_body, npad, kpad, k, gpt, ppt, scale)
  f = pl.kernel(
      body,
      out_type=[
          jax.ShapeDtypeStruct((npad,), jnp.int32),
          jax.ShapeDtypeStruct((npad,), jnp.int32),
          jax.ShapeDtypeStruct((L,), jnp.float32),
      ],
      mesh=mesh,
      compiler_params=pltpu.CompilerParams(
          needs_layout_passes=False, use_tc_tiling_on_sc=False),
      scratch_types=[
          pltpu.VMEM((npad,), jnp.float32),
          pltpu.VMEM((npad,), jnp.float32),
          pltpu.VMEM((npad,), jnp.float32),
          pltpu.VMEM((4 * kpad,), jnp.float32),
          pltpu.VMEM((4 * kpad,), jnp.float32),
          pltpu.VMEM((kpad,), jnp.int32),
          pltpu.VMEM((kpad,), jnp.float32),
          pltpu.VMEM((kpad,), jnp.float32),
          pltpu.VMEM((kpad,), jnp.float32),
          pltpu.VMEM((kpad,), jnp.float32),
          pltpu.VMEM((kpad,), jnp.float32),
          pltpu.VMEM((L,), jnp.float32),
          pltpu.VMEM((L,), jnp.int32),
          pltpu.VMEM_SHARED((NS, L), jnp.float32),
          pltpu.VMEM_SHARED((NS, L), jnp.int32),
          pltpu.VMEM((NS, L), jnp.float32),
          pltpu.VMEM((NS, L), jnp.int32),
          pltpu.VMEM((ppt,), jnp.int32),
          pltpu.VMEM((ppt,), jnp.int32),
          pltpu.VMEM((L,), jnp.float32),
      ],
  )
  ai, al, loss = f(px, py, ps, gx4, gy4, glab)
  return ai[:n], al[:n], loss[0]


# speedup vs baseline: 1.0558x; 1.0558x over previous
import functools

import jax
import jax.numpy as jnp
from jax import lax
from jax.experimental import pallas as pl
from jax.experimental.pallas import tpu as pltpu
from jax.experimental.pallas import tpu_sc as plsc

L = 16
NS = 16
BIG = 1e30
BIGI = 1 << 30


def _iota():
  return lax.iota(jnp.int32, L)


def _bcast(x, dtype):
  return jnp.full((L,), x, dtype=dtype)


def _shuf(x, sh):
  perm = jnp.bitwise_xor(_iota(), sh)
  return jnp.take_along_axis(x, perm, axis=0)


def _vmin(x):
  for sh in (1, 2, 4, 8):
    x = jnp.minimum(x, _shuf(x, sh))
  return x


def _vsum(x):
  for sh in (1, 2, 4, 8):
    x = x + _shuf(x, sh)
  return x


def _sc_body(npad, kpad, ngt, gpt, ppt, scale,
             px_h, py_h, ps_h, gx4_h, gy4_h, glab_h,
             ai_h, al_h, loss_h,
             px_v, py_v, ps_v, gx4_v, gy4_v, glab_v,
             gxv, gyv, gwv, ghv, gsv,
             mdrow_v, mirow_v, smd, smi, mdg_v, mig_v,
             ai_v, al_v, loss_v, dmasem1, dmasem2):
  sid = lax.axis_index("s")
  n_chunks = npad // L
  iot = _iota()

  cp_px = pltpu.async_copy(px_h, px_v, dmasem1)
  cp_py = pltpu.async_copy(py_h, py_v, dmasem2)
  pltpu.sync_copy(ps_h, ps_v)
  pltpu.sync_copy(gx4_h, gx4_v)
  pltpu.sync_copy(gy4_h, gy4_v)
  pltpu.sync_copy(glab_h, glab_v)

  def mm_body(c, carry):
    smin, smax = carry
    s = ps_v[pl.ds(sid * ppt + c * L, L)]
    real = s > 0.0
    smin = jnp.minimum(smin, jnp.where(real, s, BIG))
    smax = jnp.maximum(smax, jnp.where(real, s, -BIG))
    return smin, smax

  smin, smax = lax.fori_loop(
      0, ppt // L, mm_body,
      (jnp.full((L,), BIG, jnp.float32), jnp.full((L,), -BIG, jnp.float32)))
  mdrow_v[...] = jnp.where(iot < 8, _vmin(smin), _vmin(jnp.negative(smax)))
  pltpu.sync_copy(mdrow_v, smd.at[sid])
  plsc.subcore_barrier()
  pltpu.sync_copy(smd, mdg_v)
  plsc.subcore_barrier()

  def amerge_body(t, acc):
    row = plsc.load_gather(mdg_v, [_bcast(t, jnp.int32), iot])
    return jnp.minimum(acc, row)

  acc = lax.fori_loop(0, NS, amerge_body, jnp.full((L,), BIG, jnp.float32))
  smin_b = _vmin(jnp.where(iot < 8, acc, BIG))
  smax_b = jnp.negative(_vmin(jnp.where(iot < 8, BIG, acc)))

  for c in range(kpad // L):
    o = c * L
    xs0 = gx4_v[pl.ds(0 * kpad + o, L)]
    xs1 = gx4_v[pl.ds(1 * kpad + o, L)]
    xs2 = gx4_v[pl.ds(2 * kpad + o, L)]
    xs3 = gx4_v[pl.ds(3 * kpad + o, L)]
    ys0 = gy4_v[pl.ds(0 * kpad + o, L)]
    ys1 = gy4_v[pl.ds(1 * kpad + o, L)]
    ys2 = gy4_v[pl.ds(2 * kpad + o, L)]
    ys3 = gy4_v[pl.ds(3 * kpad + o, L)]
    x1 = jnp.minimum(jnp.minimum(xs0, xs1), jnp.minimum(xs2, xs3))
    x2 = jnp.maximum(jnp.maximum(xs0, xs1), jnp.maximum(xs2, xs3))
    y1 = jnp.minimum(jnp.minimum(ys0, ys1), jnp.minimum(ys2, ys3))
    y2 = jnp.maximum(jnp.maximum(ys0, ys1), jnp.maximum(ys2, ys3))
    w = jnp.maximum(x2 - x1, 1e-6)
    h = jnp.maximum(y2 - y1, 1e-6)
    p = (w * h) * (1.0 / (scale * scale))
    glvl = jnp.zeros((L,), jnp.int32)
    thr = 4.0
    for _ in range(12):
      glvl = glvl + jnp.where(p >= thr, 1, 0).astype(jnp.int32)
      thr = thr * 4.0
    gs = (jnp.int32(1) << glvl).astype(jnp.float32)
    gs = jnp.clip(gs, smin_b, smax_b)
    valid = (iot + o) < ngt
    gs = jnp.where(valid, gs, -2.0)
    gxv[pl.ds(o, L)] = (x1 + x2) * 0.5
    gyv[pl.ds(o, L)] = (y1 + y2) * 0.5
    gwv[pl.ds(o, L)] = 1.0 / w
    ghv[pl.ds(o, L)] = 1.0 / h
    gsv[pl.ds(o, L)] = gs

  cp_px.wait()
  cp_py.wait()

  def gt_body(j, carry):
    md_vec, mi_vec = carry
    g = sid * gpt + j
    gsp = _bcast(g, jnp.int32)
    gx_b = plsc.load_gather(gxv, [gsp])
    gy_b = plsc.load_gather(gyv, [gsp])
    gw_b = plsc.load_gather(gwv, [gsp])
    gh_b = plsc.load_gather(ghv, [gsp])
    gs_b = plsc.load_gather(gsv, [gsp])

    half = n_chunks // 2
    init = (jnp.full((L,), BIG, jnp.float32), jnp.zeros((L,), jnp.int32),
            jnp.full((L,), BIG, jnp.float32), jnp.zeros((L,), jnp.int32))

    @plsc.parallel_loop(0, half, 1, unroll=8, carry=init)
    def pt_loop(c, pc):
      d_a, i_a, d_b, i_b = pc
      base_a = c * L
      base_b = (c + half) * L
      pxa = px_v[pl.ds(base_a, L)]
      pya = py_v[pl.ds(base_a, L)]
      psa = ps_v[pl.ds(base_a, L)]
      pxb = px_v[pl.ds(base_b, L)]
      pyb = py_v[pl.ds(base_b, L)]
      psb = ps_v[pl.ds(base_b, L)]
      dxa = (pxa - gx_b) * gw_b
      dya = (pya - gy_b) * gh_b
      d2a = dxa * dxa + dya * dya
      dxb = (pxb - gx_b) * gw_b
      dyb = (pyb - gy_b) * gh_b
      d2b = dxb * dxb + dyb * dyb
      upd_a = (d2a < d_a) & (psa == gs_b)
      upd_b = (d2b < d_b) & (psb == gs_b)
      idx_a = iot + _bcast(base_a, jnp.int32)
      idx_b = iot + _bcast(base_b, jnp.int32)
      d_a = jnp.where(upd_a, d2a, d_a)
      i_a = jnp.where(upd_a, idx_a, i_a)
      d_b = jnp.where(upd_b, d2b, d_b)
      i_b = jnp.where(upd_b, idx_b, i_b)
      return d_a, i_a, d_b, i_b

    d_a, i_a, d_b, i_b = pt_loop
    updm = d_b < d_a
    run_d2 = jnp.where(updm, d_b, d_a)
    run_idx = jnp.where(updm, i_b, i_a)
    md_b = _vmin(run_d2)
    cand = jnp.where(run_d2 == md_b, run_idx, BIGI)
    mi_b = _vmin(cand)
    lane = iot == _bcast(j, jnp.int32)
    md_vec = jnp.where(lane, md_b, md_vec)
    mi_vec = jnp.where(lane, mi_b, mi_vec)
    return md_vec, mi_vec

  md_vec, mi_vec = lax.fori_loop(
      0, gpt, gt_body,
      (jnp.full((L,), BIG, jnp.float32), jnp.zeros((L,), jnp.int32)))

  mdrow_v[...] = md_vec
  mirow_v[...] = mi_vec
  pltpu.sync_copy(mdrow_v, smd.at[sid])
  pltpu.sync_copy(mirow_v, smi.at[sid])
  plsc.subcore_barrier()
  pltpu.sync_copy(smd, mdg_v)
  pltpu.sync_copy(smi, mig_v)

  for c in range(ppt // L):
    ai_v[pl.ds(c * L, L)] = jnp.zeros((L,), jnp.int32)
    al_v[pl.ds(c * L, L)] = jnp.zeros((L,), jnp.int32)

  base = sid * ppt
  base_b = _bcast(base, jnp.int32)
  lsum = jnp.zeros((L,), jnp.float32)
  lcnt = jnp.zeros((L,), jnp.float32)
  for r in range(gpt):
    k_vec = iot * gpt + r
    rsp = _bcast(r, jnp.int32)
    md_col = plsc.load_gather(mdg_v, [iot, rsp])
    mi_col = plsc.load_gather(mig_v, [iot, rsp])
    valid_k = md_col < (BIG * 0.5)

    def kill_body(jt, killed):
      jtsp = _bcast(jt, jnp.int32)
      k = killed
      for jj in range(gpt):
        jjsp = _bcast(jj, jnp.int32)
        md_j = plsc.load_gather(mdg_v, [jtsp, jjsp])
        mi_j = plsc.load_gather(mig_v, [jtsp, jjsp])
        g_j = jt * gpt + jj
        g_j_b = _bcast(g_j, jnp.int32)
        better = (md_j < md_col) | ((md_j == md_col) & (g_j_b < k_vec))
        k = k | ((md_j < (BIG * 0.5)) & (mi_j == mi_col) & better)
      return k

    killed = lax.fori_loop(0, NS, kill_body, jnp.zeros((L,), jnp.bool_))
    win = valid_k & jnp.logical_not(killed)

    lsum = lsum + jnp.where(win, md_col, 0.0)
    lcnt = lcnt + jnp.where(win, 1.0, 0.0)

    mine = win & (mi_col >= base_b) & (mi_col < base_b + ppt)
    loc = mi_col - base_b
    loc = jnp.where(mine, loc, 0)
    plsc.store_scatter(ai_v, [loc], k_vec + 1, mask=mine)
    labs = plsc.load_gather(glab_v, [k_vec])
    plsc.store_scatter(al_v, [loc], labs, mask=mine)

  total = _vsum(lsum)
  cnt = _vsum(lcnt)
  loss_v[...] = total / jnp.maximum(cnt, 1.0)

  pltpu.sync_copy(ai_v, ai_h.at[pl.ds(base, ppt)])
  pltpu.sync_copy(al_v, al_h.at[pl.ds(base, ppt)])

  @pl.when(sid == 0)
  def _():
    pltpu.sync_copy(loss_v, loss_h)


@jax.jit
def kernel(points, gt_obboxes, gt_labels):
  n = points.shape[0]
  k = gt_obboxes.shape[0]
  scale = 4.0

  npad = ((n + L * NS - 1) // (L * NS)) * (L * NS)
  ppt = npad // NS
  gpt = (k + NS - 1) // NS
  kpad = gpt * NS

  px = jnp.pad(points[:, 0], (0, npad - n))
  py = jnp.pad(points[:, 1], (0, npad - n))
  ps = jnp.pad(points[:, 2], (0, npad - n), constant_values=-1.0)
  gx4 = jnp.pad(gt_obboxes[:, 0::2].T, ((0, 0), (0, kpad - k)),
                constant_values=1.0).reshape(-1)
  gy4 = jnp.pad(gt_obboxes[:, 1::2].T, ((0, 0), (0, kpad - k)),
                constant_values=1.0).reshape(-1)
  glab = jnp.pad(gt_labels.astype(jnp.int32), (0, kpad - k))

  mesh = plsc.VectorSubcoreMesh(
      core_axis_name="c", subcore_axis_name="s", num_cores=1, num_subcores=NS)

  body = functools.partial(_sc_body, npad, kpad, k, gpt, ppt, scale)
  f = pl.kernel(
      body,
      out_type=[
          jax.ShapeDtypeStruct((npad,), jnp.int32),
          jax.ShapeDtypeStruct((npad,), jnp.int32),
          jax.ShapeDtypeStruct((L,), jnp.float32),
      ],
      mesh=mesh,
      compiler_params=pltpu.CompilerParams(
          needs_layout_passes=False, use_tc_tiling_on_sc=False),
      scratch_types=[
          pltpu.VMEM((npad,), jnp.float32),
          pltpu.VMEM((npad,), jnp.float32),
          pltpu.VMEM((npad,), jnp.float32),
          pltpu.VMEM((4 * kpad,), jnp.float32),
          pltpu.VMEM((4 * kpad,), jnp.float32),
          pltpu.VMEM((kpad,), jnp.int32),
          pltpu.VMEM((kpad,), jnp.float32),
          pltpu.VMEM((kpad,), jnp.float32),
          pltpu.VMEM((kpad,), jnp.float32),
          pltpu.VMEM((kpad,), jnp.float32),
          pltpu.VMEM((kpad,), jnp.float32),
          pltpu.VMEM((L,), jnp.float32),
          pltpu.VMEM((L,), jnp.int32),
          pltpu.VMEM_SHARED((NS, L), jnp.float32),
          pltpu.VMEM_SHARED((NS, L), jnp.int32),
          pltpu.VMEM((NS, L), jnp.float32),
          pltpu.VMEM((NS, L), jnp.int32),
          pltpu.VMEM((ppt,), jnp.int32),
          pltpu.VMEM((ppt,), jnp.int32),
          pltpu.VMEM((L,), jnp.float32),
          pltpu.SemaphoreType.DMA,
          pltpu.SemaphoreType.DMA,
      ],
  )
  ai, al, loss = f(px, py, ps, gx4, gy4, glab)
  return ai[:n], al[:n], loss[0]


# speedup vs baseline: 1.1173x; 1.0582x over previous
import functools

import jax
import jax.numpy as jnp
from jax import lax
from jax.experimental import pallas as pl
from jax.experimental.pallas import tpu as pltpu
from jax.experimental.pallas import tpu_sc as plsc

L = 16
NS = 16
BIG = 1e30
BIGI = 1 << 30


def _iota():
  return lax.iota(jnp.int32, L)


def _bcast(x, dtype):
  return jnp.full((L,), x, dtype=dtype)


def _shuf(x, sh):
  perm = jnp.bitwise_xor(_iota(), sh)
  return jnp.take_along_axis(x, perm, axis=0)


def _vmin(x):
  for sh in (1, 2, 4, 8):
    x = jnp.minimum(x, _shuf(x, sh))
  return x


def _vsum(x):
  for sh in (1, 2, 4, 8):
    x = x + _shuf(x, sh)
  return x


def _sc_body(npad, kpad, ngt, gpt, ppt, scale,
             px_h, py_h, ps_h, gx4_h, gy4_h, glab_h,
             ai_h, al_h, loss_h,
             px_v, py_v, ps_v, gx4_v, gy4_v, glab_v,
             gxv, gyv, gwv, ghv, gsv,
             mdrow_v, mirow_v, smd, smi, mdg_v, mig_v,
             ai_v, al_v, loss_v, dmasem1, dmasem2):
  sid = lax.axis_index("s")
  n_chunks = npad // L
  iot = _iota()

  cp_px = pltpu.async_copy(px_h, px_v, dmasem1)
  cp_py = pltpu.async_copy(py_h, py_v, dmasem2)
  pltpu.sync_copy(ps_h, ps_v)
  pltpu.sync_copy(gx4_h, gx4_v)
  pltpu.sync_copy(gy4_h, gy4_v)
  pltpu.sync_copy(glab_h, glab_v)

  def mm_body(c, carry):
    smin, smax = carry
    s = ps_v[pl.ds(sid * ppt + c * L, L)]
    real = s > 0.0
    smin = jnp.minimum(smin, jnp.where(real, s, BIG))
    smax = jnp.maximum(smax, jnp.where(real, s, -BIG))
    return smin, smax

  smin, smax = lax.fori_loop(
      0, ppt // L, mm_body,
      (jnp.full((L,), BIG, jnp.float32), jnp.full((L,), -BIG, jnp.float32)))
  mdrow_v[...] = jnp.where(iot < 8, _vmin(smin), _vmin(jnp.negative(smax)))
  pltpu.sync_copy(mdrow_v, smd.at[sid])
  plsc.subcore_barrier()
  pltpu.sync_copy(smd, mdg_v)
  plsc.subcore_barrier()

  def amerge_body(t, acc):
    row = plsc.load_gather(mdg_v, [_bcast(t, jnp.int32), iot])
    return jnp.minimum(acc, row)

  acc = lax.fori_loop(0, NS, amerge_body, jnp.full((L,), BIG, jnp.float32))
  smin_b = _vmin(jnp.where(iot < 8, acc, BIG))
  smax_b = jnp.negative(_vmin(jnp.where(iot < 8, BIG, acc)))

  for c in range(kpad // L):
    o = c * L
    xs0 = gx4_v[pl.ds(0 * kpad + o, L)]
    xs1 = gx4_v[pl.ds(1 * kpad + o, L)]
    xs2 = gx4_v[pl.ds(2 * kpad + o, L)]
    xs3 = gx4_v[pl.ds(3 * kpad + o, L)]
    ys0 = gy4_v[pl.ds(0 * kpad + o, L)]
    ys1 = gy4_v[pl.ds(1 * kpad + o, L)]
    ys2 = gy4_v[pl.ds(2 * kpad + o, L)]
    ys3 = gy4_v[pl.ds(3 * kpad + o, L)]
    x1 = jnp.minimum(jnp.minimum(xs0, xs1), jnp.minimum(xs2, xs3))
    x2 = jnp.maximum(jnp.maximum(xs0, xs1), jnp.maximum(xs2, xs3))
    y1 = jnp.minimum(jnp.minimum(ys0, ys1), jnp.minimum(ys2, ys3))
    y2 = jnp.maximum(jnp.maximum(ys0, ys1), jnp.maximum(ys2, ys3))
    w = jnp.maximum(x2 - x1, 1e-6)
    h = jnp.maximum(y2 - y1, 1e-6)
    p = (w * h) * (1.0 / (scale * scale))
    glvl = jnp.zeros((L,), jnp.int32)
    thr = 4.0
    for _ in range(12):
      glvl = glvl + jnp.where(p >= thr, 1, 0).astype(jnp.int32)
      thr = thr * 4.0
    gs = (jnp.int32(1) << glvl).astype(jnp.float32)
    gs = jnp.clip(gs, smin_b, smax_b)
    valid = (iot + o) < ngt
    gs = jnp.where(valid, gs, -2.0)
    gxv[pl.ds(o, L)] = (x1 + x2) * 0.5
    gyv[pl.ds(o, L)] = (y1 + y2) * 0.5
    gwv[pl.ds(o, L)] = 1.0 / w
    ghv[pl.ds(o, L)] = 1.0 / h
    gsv[pl.ds(o, L)] = gs

  cp_px.wait()
  cp_py.wait()

  def gt_body(j, carry):
    md_vec, mi_vec = carry
    g = sid * gpt + j
    gsp = _bcast(g, jnp.int32)
    gx_b = plsc.load_gather(gxv, [gsp])
    gy_b = plsc.load_gather(gyv, [gsp])
    gw_b = plsc.load_gather(gwv, [gsp])
    gh_b = plsc.load_gather(ghv, [gsp])
    gs_b = plsc.load_gather(gsv, [gsp])

    half = n_chunks // 2
    init = (jnp.full((L,), BIG, jnp.float32), jnp.zeros((L,), jnp.int32),
            jnp.full((L,), BIG, jnp.float32), jnp.zeros((L,), jnp.int32))

    @plsc.parallel_loop(0, half, 1, unroll=4, carry=init)
    def pt_loop(c, pc):
      d_a, i_a, d_b, i_b = pc
      base_a = c * L
      base_b = (c + half) * L
      pxa = px_v[pl.ds(base_a, L)]
      pya = py_v[pl.ds(base_a, L)]
      psa = ps_v[pl.ds(base_a, L)]
      pxb = px_v[pl.ds(base_b, L)]
      pyb = py_v[pl.ds(base_b, L)]
      psb = ps_v[pl.ds(base_b, L)]
      dxa = (pxa - gx_b) * gw_b
      dya = (pya - gy_b) * gh_b
      d2a = dxa * dxa + dya * dya
      dxb = (pxb - gx_b) * gw_b
      dyb = (pyb - gy_b) * gh_b
      d2b = dxb * dxb + dyb * dyb
      upd_a = (d2a < d_a) & (psa == gs_b)
      upd_b = (d2b < d_b) & (psb == gs_b)
      idx_a = iot + _bcast(base_a, jnp.int32)
      idx_b = iot + _bcast(base_b, jnp.int32)
      d_a = jnp.where(upd_a, d2a, d_a)
      i_a = jnp.where(upd_a, idx_a, i_a)
      d_b = jnp.where(upd_b, d2b, d_b)
      i_b = jnp.where(upd_b, idx_b, i_b)
      return d_a, i_a, d_b, i_b

    d_a, i_a, d_b, i_b = pt_loop
    updm = d_b < d_a
    run_d2 = jnp.where(updm, d_b, d_a)
    run_idx = jnp.where(updm, i_b, i_a)
    md_b = _vmin(run_d2)
    cand = jnp.where(run_d2 == md_b, run_idx, BIGI)
    mi_b = _vmin(cand)
    lane = iot == _bcast(j, jnp.int32)
    md_vec = jnp.where(lane, md_b, md_vec)
    mi_vec = jnp.where(lane, mi_b, mi_vec)
    return md_vec, mi_vec

  md_vec, mi_vec = lax.fori_loop(
      0, gpt, gt_body,
      (jnp.full((L,), BIG, jnp.float32), jnp.zeros((L,), jnp.int32)))

  mdrow_v[...] = md_vec
  mirow_v[...] = mi_vec
  pltpu.sync_copy(mdrow_v, smd.at[sid])
  pltpu.sync_copy(mirow_v, smi.at[sid])
  plsc.subcore_barrier()
  pltpu.sync_copy(smd, mdg_v)
  pltpu.sync_copy(smi, mig_v)

  for c in range(ppt // L):
    ai_v[pl.ds(c * L, L)] = jnp.zeros((L,), jnp.int32)
    al_v[pl.ds(c * L, L)] = jnp.zeros((L,), jnp.int32)

  base = sid * ppt
  base_b = _bcast(base, jnp.int32)
  lsum = jnp.zeros((L,), jnp.float32)
  lcnt = jnp.zeros((L,), jnp.float32)
  for r in range(gpt):
    k_vec = iot * gpt + r
    rsp = _bcast(r, jnp.int32)
    md_col = plsc.load_gather(mdg_v, [iot, rsp])
    mi_col = plsc.load_gather(mig_v, [iot, rsp])
    valid_k = md_col < (BIG * 0.5)

    def kill_body(jt, killed):
      jtsp = _bcast(jt, jnp.int32)
      k = killed
      for jj in range(gpt):
        jjsp = _bcast(jj, jnp.int32)
        md_j = plsc.load_gather(mdg_v, [jtsp, jjsp])
        mi_j = plsc.load_gather(mig_v, [jtsp, jjsp])
        g_j = jt * gpt + jj
        g_j_b = _bcast(g_j, jnp.int32)
        better = (md_j < md_col) | ((md_j == md_col) & (g_j_b < k_vec))
        k = k | ((md_j < (BIG * 0.5)) & (mi_j == mi_col) & better)
      return k

    killed = lax.fori_loop(0, NS, kill_body, jnp.zeros((L,), jnp.bool_))
    win = valid_k & jnp.logical_not(killed)

    lsum = lsum + jnp.where(win, md_col, 0.0)
    lcnt = lcnt + jnp.where(win, 1.0, 0.0)

    mine = win & (mi_col >= base_b) & (mi_col < base_b + ppt)
    loc = mi_col - base_b
    loc = jnp.where(mine, loc, 0)
    plsc.store_scatter(ai_v, [loc], k_vec + 1, mask=mine)
    labs = plsc.load_gather(glab_v, [k_vec])
    plsc.store_scatter(al_v, [loc], labs, mask=mine)

  total = _vsum(lsum)
  cnt = _vsum(lcnt)
  loss_v[...] = total / jnp.maximum(cnt, 1.0)

  pltpu.sync_copy(ai_v, ai_h.at[pl.ds(base, ppt)])
  pltpu.sync_copy(al_v, al_h.at[pl.ds(base, ppt)])

  @pl.when(sid == 0)
  def _():
    pltpu.sync_copy(loss_v, loss_h)


@jax.jit
def kernel(points, gt_obboxes, gt_labels):
  n = points.shape[0]
  k = gt_obboxes.shape[0]
  scale = 4.0

  npad = ((n + L * NS - 1) // (L * NS)) * (L * NS)
  ppt = npad // NS
  gpt = (k + NS - 1) // NS
  kpad = gpt * NS

  px = jnp.pad(points[:, 0], (0, npad - n))
  py = jnp.pad(points[:, 1], (0, npad - n))
  ps = jnp.pad(points[:, 2], (0, npad - n), constant_values=-1.0)
  gx4 = jnp.pad(gt_obboxes[:, 0::2].T, ((0, 0), (0, kpad - k)),
                constant_values=1.0).reshape(-1)
  gy4 = jnp.pad(gt_obboxes[:, 1::2].T, ((0, 0), (0, kpad - k)),
                constant_values=1.0).reshape(-1)
  glab = jnp.pad(gt_labels.astype(jnp.int32), (0, kpad - k))

  mesh = plsc.VectorSubcoreMesh(
      core_axis_name="c", subcore_axis_name="s", num_cores=1, num_subcores=NS)

  body = functools.partial(_sc_body, npad, kpad, k, gpt, ppt, scale)
  f = pl.kernel(
      body,
      out_type=[
          jax.ShapeDtypeStruct((npad,), jnp.int32),
          jax.ShapeDtypeStruct((npad,), jnp.int32),
          jax.ShapeDtypeStruct((L,), jnp.float32),
      ],
      mesh=mesh,
      compiler_params=pltpu.CompilerParams(
          needs_layout_passes=False, use_tc_tiling_on_sc=False),
      scratch_types=[
          pltpu.VMEM((npad,), jnp.float32),
          pltpu.VMEM((npad,), jnp.float32),
          pltpu.VMEM((npad,), jnp.float32),
          pltpu.VMEM((4 * kpad,), jnp.float32),
          pltpu.VMEM((4 * kpad,), jnp.float32),
          pltpu.VMEM((kpad,), jnp.int32),
          pltpu.VMEM((kpad,), jnp.float32),
          pltpu.VMEM((kpad,), jnp.float32),
          pltpu.VMEM((kpad,), jnp.float32),
          pltpu.VMEM((kpad,), jnp.float32),
          pltpu.VMEM((kpad,), jnp.float32),
          pltpu.VMEM((L,), jnp.float32),
          pltpu.VMEM((L,), jnp.int32),
          pltpu.VMEM_SHARED((NS, L), jnp.float32),
          pltpu.VMEM_SHARED((NS, L), jnp.int32),
          pltpu.VMEM((NS, L), jnp.float32),
          pltpu.VMEM((NS, L), jnp.int32),
          pltpu.VMEM((ppt,), jnp.int32),
          pltpu.VMEM((ppt,), jnp.int32),
          pltpu.VMEM((L,), jnp.float32),
          pltpu.SemaphoreType.DMA,
          pltpu.SemaphoreType.DMA,
      ],
  )
  ai, al, loss = f(px, py, ps, gx4, gy4, glab)
  return ai[:n], al[:n], loss[0]


# speedup vs baseline: 1.1573x; 1.0358x over previous
import functools

import jax
import jax.numpy as jnp
from jax import lax
from jax.experimental import pallas as pl
from jax.experimental.pallas import tpu as pltpu
from jax.experimental.pallas import tpu_sc as plsc

L = 16
NS = 16
BIG = 1e30
BIGI = 1 << 30


def _iota():
  return lax.iota(jnp.int32, L)


def _bcast(x, dtype):
  return jnp.full((L,), x, dtype=dtype)


def _shuf(x, sh):
  perm = jnp.bitwise_xor(_iota(), sh)
  return jnp.take_along_axis(x, perm, axis=0)


def _vmin(x):
  for sh in (1, 2, 4, 8):
    x = jnp.minimum(x, _shuf(x, sh))
  return x


def _vsum(x):
  for sh in (1, 2, 4, 8):
    x = x + _shuf(x, sh)
  return x


def _sc_body(npad, kpad, ngt, gpt, ppt, scale,
             px_h, py_h, ps_h, gx4_h, gy4_h, glab_h,
             ai_h, al_h, loss_h,
             px_v, py_v, ps_v, gx4_v, gy4_v, glab_v,
             gxv, gyv, gwv, ghv, gsv,
             mdrow_v, mirow_v, smd, smi, mdg_v, mig_v,
             ai_v, al_v, loss_v, dmasem1, dmasem2):
  sid = lax.axis_index("s")
  n_chunks = npad // L
  iot = _iota()

  cp_px = pltpu.async_copy(px_h, px_v, dmasem1)
  cp_py = pltpu.async_copy(py_h, py_v, dmasem2)
  pltpu.sync_copy(ps_h, ps_v)
  pltpu.sync_copy(gx4_h, gx4_v)
  pltpu.sync_copy(gy4_h, gy4_v)
  pltpu.sync_copy(glab_h, glab_v)

  def mm_body(c, carry):
    smin, smax = carry
    s = ps_v[pl.ds(sid * ppt + c * L, L)]
    real = s > 0.0
    smin = jnp.minimum(smin, jnp.where(real, s, BIG))
    smax = jnp.maximum(smax, jnp.where(real, s, -BIG))
    return smin, smax

  smin, smax = lax.fori_loop(
      0, ppt // L, mm_body,
      (jnp.full((L,), BIG, jnp.float32), jnp.full((L,), -BIG, jnp.float32)))
  mdrow_v[...] = jnp.where(iot < 8, _vmin(smin), _vmin(jnp.negative(smax)))
  pltpu.sync_copy(mdrow_v, smd.at[sid])
  plsc.subcore_barrier()
  pltpu.sync_copy(smd, mdg_v)
  plsc.subcore_barrier()

  def amerge_body(t, acc):
    row = plsc.load_gather(mdg_v, [_bcast(t, jnp.int32), iot])
    return jnp.minimum(acc, row)

  acc = lax.fori_loop(0, NS, amerge_body, jnp.full((L,), BIG, jnp.float32))
  smin_b = _vmin(jnp.where(iot < 8, acc, BIG))
  smax_b = jnp.negative(_vmin(jnp.where(iot < 8, BIG, acc)))

  for c in range(kpad // L):
    o = c * L
    xs0 = gx4_v[pl.ds(0 * kpad + o, L)]
    xs1 = gx4_v[pl.ds(1 * kpad + o, L)]
    xs2 = gx4_v[pl.ds(2 * kpad + o, L)]
    xs3 = gx4_v[pl.ds(3 * kpad + o, L)]
    ys0 = gy4_v[pl.ds(0 * kpad + o, L)]
    ys1 = gy4_v[pl.ds(1 * kpad + o, L)]
    ys2 = gy4_v[pl.ds(2 * kpad + o, L)]
    ys3 = gy4_v[pl.ds(3 * kpad + o, L)]
    x1 = jnp.minimum(jnp.minimum(xs0, xs1), jnp.minimum(xs2, xs3))
    x2 = jnp.maximum(jnp.maximum(xs0, xs1), jnp.maximum(xs2, xs3))
    y1 = jnp.minimum(jnp.minimum(ys0, ys1), jnp.minimum(ys2, ys3))
    y2 = jnp.maximum(jnp.maximum(ys0, ys1), jnp.maximum(ys2, ys3))
    w = jnp.maximum(x2 - x1, 1e-6)
    h = jnp.maximum(y2 - y1, 1e-6)
    p = (w * h) * (1.0 / (scale * scale))
    glvl = jnp.zeros((L,), jnp.int32)
    thr = 4.0
    for _ in range(12):
      glvl = glvl + jnp.where(p >= thr, 1, 0).astype(jnp.int32)
      thr = thr * 4.0
    gs = (jnp.int32(1) << glvl).astype(jnp.float32)
    gs = jnp.clip(gs, smin_b, smax_b)
    valid = (iot + o) < ngt
    gs = jnp.where(valid, gs, -2.0)
    gxv[pl.ds(o, L)] = (x1 + x2) * 0.5
    gyv[pl.ds(o, L)] = (y1 + y2) * 0.5
    gwv[pl.ds(o, L)] = 1.0 / w
    ghv[pl.ds(o, L)] = 1.0 / h
    gsv[pl.ds(o, L)] = gs

  cp_px.wait()
  cp_py.wait()

  def gt_body(j, carry):
    md_vec, mi_vec = carry
    g = sid * gpt + j
    gsp = _bcast(g, jnp.int32)
    gx_b = plsc.load_gather(gxv, [gsp])
    gy_b = plsc.load_gather(gyv, [gsp])
    gw_b = plsc.load_gather(gwv, [gsp])
    gh_b = plsc.load_gather(ghv, [gsp])
    gs_b = plsc.load_gather(gsv, [gsp])

    half = n_chunks // 2
    init = (jnp.full((L,), BIG, jnp.float32), jnp.zeros((L,), jnp.int32),
            jnp.full((L,), BIG, jnp.float32), jnp.zeros((L,), jnp.int32))

    @plsc.parallel_loop(0, half, 1, unroll=4, carry=init)
    def pt_loop(c, pc):
      d_a, i_a, d_b, i_b = pc
      base_a = c * L
      base_b = (c + half) * L
      pxa = px_v[pl.ds(base_a, L)]
      pya = py_v[pl.ds(base_a, L)]
      psa = ps_v[pl.ds(base_a, L)]
      pxb = px_v[pl.ds(base_b, L)]
      pyb = py_v[pl.ds(base_b, L)]
      psb = ps_v[pl.ds(base_b, L)]
      dxa = (pxa - gx_b) * gw_b
      dya = (pya - gy_b) * gh_b
      d2a = dxa * dxa + dya * dya
      dxb = (pxb - gx_b) * gw_b
      dyb = (pyb - gy_b) * gh_b
      d2b = dxb * dxb + dyb * dyb
      upd_a = (d2a < d_a) & (psa == gs_b)
      upd_b = (d2b < d_b) & (psb == gs_b)
      csp = _bcast(c, jnp.int32)
      d_a = jnp.where(upd_a, d2a, d_a)
      i_a = jnp.where(upd_a, csp, i_a)
      d_b = jnp.where(upd_b, d2b, d_b)
      i_b = jnp.where(upd_b, csp, i_b)
      return d_a, i_a, d_b, i_b

    d_a, i_a, d_b, i_b = pt_loop
    ia_full = i_a * L + iot
    ib_full = (i_b + half) * L + iot
    updm = d_b < d_a
    run_d2 = jnp.where(updm, d_b, d_a)
    run_idx = jnp.where(updm, ib_full, ia_full)
    md_b = _vmin(run_d2)
    cand = jnp.where(run_d2 == md_b, run_idx, BIGI)
    mi_b = _vmin(cand)
    lane = iot == _bcast(j, jnp.int32)
    md_vec = jnp.where(lane, md_b, md_vec)
    mi_vec = jnp.where(lane, mi_b, mi_vec)
    return md_vec, mi_vec

  md_vec, mi_vec = lax.fori_loop(
      0, gpt, gt_body,
      (jnp.full((L,), BIG, jnp.float32), jnp.zeros((L,), jnp.int32)))

  mdrow_v[...] = md_vec
  mirow_v[...] = mi_vec
  pltpu.sync_copy(mdrow_v, smd.at[sid])
  pltpu.sync_copy(mirow_v, smi.at[sid])
  plsc.subcore_barrier()
  pltpu.sync_copy(smd, mdg_v)
  pltpu.sync_copy(smi, mig_v)

  for c in range(ppt // L):
    ai_v[pl.ds(c * L, L)] = jnp.zeros((L,), jnp.int32)
    al_v[pl.ds(c * L, L)] = jnp.zeros((L,), jnp.int32)

  base = sid * ppt
  base_b = _bcast(base, jnp.int32)
  lsum = jnp.zeros((L,), jnp.float32)
  lcnt = jnp.zeros((L,), jnp.float32)
  for r in range(gpt):
    k_vec = iot * gpt + r
    rsp = _bcast(r, jnp.int32)
    md_col = plsc.load_gather(mdg_v, [iot, rsp])
    mi_col = plsc.load_gather(mig_v, [iot, rsp])
    valid_k = md_col < (BIG * 0.5)

    def kill_body(jt, killed):
      jtsp = _bcast(jt, jnp.int32)
      k = killed
      for jj in range(gpt):
        jjsp = _bcast(jj, jnp.int32)
        md_j = plsc.load_gather(mdg_v, [jtsp, jjsp])
        mi_j = plsc.load_gather(mig_v, [jtsp, jjsp])
        g_j = jt * gpt + jj
        g_j_b = _bcast(g_j, jnp.int32)
        better = (md_j < md_col) | ((md_j == md_col) & (g_j_b < k_vec))
        k = k | ((md_j < (BIG * 0.5)) & (mi_j == mi_col) & better)
      return k

    killed = lax.fori_loop(0, NS, kill_body, jnp.zeros((L,), jnp.bool_))
    win = valid_k & jnp.logical_not(killed)

    lsum = lsum + jnp.where(win, md_col, 0.0)
    lcnt = lcnt + jnp.where(win, 1.0, 0.0)

    mine = win & (mi_col >= base_b) & (mi_col < base_b + ppt)
    loc = mi_col - base_b
    loc = jnp.where(mine, loc, 0)
    plsc.store_scatter(ai_v, [loc], k_vec + 1, mask=mine)
    labs = plsc.load_gather(glab_v, [k_vec])
    plsc.store_scatter(al_v, [loc], labs, mask=mine)

  total = _vsum(lsum)
  cnt = _vsum(lcnt)
  loss_v[...] = total / jnp.maximum(cnt, 1.0)

  pltpu.sync_copy(ai_v, ai_h.at[pl.ds(base, ppt)])
  pltpu.sync_copy(al_v, al_h.at[pl.ds(base, ppt)])

  @pl.when(sid == 0)
  def _():
    pltpu.sync_copy(loss_v, loss_h)


@jax.jit
def kernel(points, gt_obboxes, gt_labels):
  n = points.shape[0]
  k = gt_obboxes.shape[0]
  scale = 4.0

  npad = ((n + L * NS - 1) // (L * NS)) * (L * NS)
  ppt = npad // NS
  gpt = (k + NS - 1) // NS
  kpad = gpt * NS

  px = jnp.pad(points[:, 0], (0, npad - n))
  py = jnp.pad(points[:, 1], (0, npad - n))
  ps = jnp.pad(points[:, 2], (0, npad - n), constant_values=-1.0)
  gx4 = jnp.pad(gt_obboxes[:, 0::2].T, ((0, 0), (0, kpad - k)),
                constant_values=1.0).reshape(-1)
  gy4 = jnp.pad(gt_obboxes[:, 1::2].T, ((0, 0), (0, kpad - k)),
                constant_values=1.0).reshape(-1)
  glab = jnp.pad(gt_labels.astype(jnp.int32), (0, kpad - k))

  mesh = plsc.VectorSubcoreMesh(
      core_axis_name="c", subcore_axis_name="s", num_cores=1, num_subcores=NS)

  body = functools.partial(_sc_body, npad, kpad, k, gpt, ppt, scale)
  f = pl.kernel(
      body,
      out_type=[
          jax.ShapeDtypeStruct((npad,), jnp.int32),
          jax.ShapeDtypeStruct((npad,), jnp.int32),
          jax.ShapeDtypeStruct((L,), jnp.float32),
      ],
      mesh=mesh,
      compiler_params=pltpu.CompilerParams(
          needs_layout_passes=False, use_tc_tiling_on_sc=False),
      scratch_types=[
          pltpu.VMEM((npad,), jnp.float32),
          pltpu.VMEM((npad,), jnp.float32),
          pltpu.VMEM((npad,), jnp.float32),
          pltpu.VMEM((4 * kpad,), jnp.float32),
          pltpu.VMEM((4 * kpad,), jnp.float32),
          pltpu.VMEM((kpad,), jnp.int32),
          pltpu.VMEM((kpad,), jnp.float32),
          pltpu.VMEM((kpad,), jnp.float32),
          pltpu.VMEM((kpad,), jnp.float32),
          pltpu.VMEM((kpad,), jnp.float32),
          pltpu.VMEM((kpad,), jnp.float32),
          pltpu.VMEM((L,), jnp.float32),
          pltpu.VMEM((L,), jnp.int32),
          pltpu.VMEM_SHARED((NS, L), jnp.float32),
          pltpu.VMEM_SHARED((NS, L), jnp.int32),
          pltpu.VMEM((NS, L), jnp.float32),
          pltpu.VMEM((NS, L), jnp.int32),
          pltpu.VMEM((ppt,), jnp.int32),
          pltpu.VMEM((ppt,), jnp.int32),
          pltpu.VMEM((L,), jnp.float32),
          pltpu.SemaphoreType.DMA,
          pltpu.SemaphoreType.DMA,
      ],
  )
  ai, al, loss = f(px, py, ps, gx4, gy4, glab)
  return ai[:n], al[:n], loss[0]
